# baseline (speedup 1.0000x reference)
import functools

import jax
import jax.numpy as jnp
import numpy as np
from jax.experimental import pallas as pl
from jax.experimental.pallas import tpu as pltpu

_E = 8
_DM = 3072
_DH = 128
_DO = 10
_BLK = 256


def _moe_body(x_ref, gw_ref, gb_ref, w1_ref, b1_ref, w2_ref, b2_ref,
              emat_ref, out_ref):
    x = x_ref[...]
    g = jnp.dot(x, gw_ref[...],
                preferred_element_type=jnp.float32) + gb_ref[...]
    idx = jax.lax.broadcasted_iota(jnp.int32, g.shape, 1)
    m1 = jnp.max(g, axis=-1, keepdims=True)
    e1 = jnp.min(jnp.where(g == m1, idx, _E), axis=-1, keepdims=True)
    gm = jnp.where(idx == e1, -jnp.inf, g)
    m2 = jnp.max(gm, axis=-1, keepdims=True)
    e2 = jnp.min(jnp.where(gm == m2, idx, _E), axis=-1, keepdims=True)
    r = jnp.exp(m2 - m1)
    w_top = 1.0 / (1.0 + r)
    w_sec = r / (1.0 + r)
    c = jnp.where(idx == e1, w_top, 0.0) + jnp.where(idx == e2, w_sec, 0.0)
    h = jnp.dot(x.astype(jnp.bfloat16), w1_ref[...],
                preferred_element_type=jnp.float32)
    h = jnp.maximum(h + b1_ref[...], 0.0)
    cw = jnp.dot(c, emat_ref[...], preferred_element_type=jnp.float32)
    hw = h * cw
    out = jnp.dot(hw, w2_ref[...], preferred_element_type=jnp.float32)
    out_ref[...] = out + jnp.dot(c, b2_ref[...],
                                 preferred_element_type=jnp.float32)


@functools.partial(jax.jit, static_argnames=("interpret",))
def _moe(x, gate_W, gate_b, W1, b1, W2, b2, interpret=False):
    B = x.shape[0]
    w1cat = W1.astype(jnp.bfloat16).transpose(1, 0, 2).reshape(_DM, _E * _DH)
    b1cat = b1.reshape(1, _E * _DH)
    w2cat = W2.reshape(_E * _DH, _DO)
    emat = jnp.asarray(np.kron(np.eye(_E, dtype=np.float32),
                               np.ones((1, _DH), np.float32)))
    grid = (B // _BLK,)
    out = pl.pallas_call(
        _moe_body,
        grid=grid,
        in_specs=[
            pl.BlockSpec((_BLK, _DM), lambda i: (i, 0)),
            pl.BlockSpec((_DM, _E), lambda i: (0, 0)),
            pl.BlockSpec((1, _E), lambda i: (0, 0)),
            pl.BlockSpec((_DM, _E * _DH), lambda i: (0, 0)),
            pl.BlockSpec((1, _E * _DH), lambda i: (0, 0)),
            pl.BlockSpec((_E * _DH, _DO), lambda i: (0, 0)),
            pl.BlockSpec((_E, _DO), lambda i: (0, 0)),
            pl.BlockSpec((_E, _E * _DH), lambda i: (0, 0)),
        ],
        out_specs=pl.BlockSpec((_BLK, _DO), lambda i: (i, 0)),
        out_shape=jax.ShapeDtypeStruct((B, _DO), jnp.float32),
        interpret=interpret,
    )(x, gate_W, gate_b.reshape(1, _E), w1cat, b1cat, w2cat, b2, emat)
    return out


def kernel(x, gate_W, gate_b, W1, b1, W2, b2):
    return _moe(x, gate_W, gate_b, W1, b1, W2, b2)


# speedup vs baseline: 1.0212x; 1.0212x over previous
import functools

import jax
import jax.numpy as jnp
import numpy as np
from jax.experimental import pallas as pl
from jax.experimental.pallas import tpu as pltpu

_E = 8
_DM = 3072
_DH = 128
_DO = 10
_BLK = 512


def _relayout_body(w1_ref, out_ref):
    out_ref[...] = w1_ref[0].astype(jnp.bfloat16)


def _relayout_w1(W1):
    return pl.pallas_call(
        _relayout_body,
        grid=(_E,),
        in_specs=[pl.BlockSpec((1, _DM, _DH), lambda e: (e, 0, 0))],
        out_specs=pl.BlockSpec((_DM, _DH), lambda e: (0, e)),
        out_shape=jax.ShapeDtypeStruct((_DM, _E * _DH), jnp.bfloat16),
    )(W1)


def _moe_body(x_ref, gw_ref, gb_ref, w1_ref, b1_ref, w2_ref, b2_ref,
              emat_ref, out_ref):
    x = x_ref[...]
    g = jnp.dot(x, gw_ref[...],
                preferred_element_type=jnp.float32) + gb_ref[...]
    idx = jax.lax.broadcasted_iota(jnp.int32, g.shape, 1)
    m1 = jnp.max(g, axis=-1, keepdims=True)
    e1 = jnp.min(jnp.where(g == m1, idx, _E), axis=-1, keepdims=True)
    gm = jnp.where(idx == e1, -jnp.inf, g)
    m2 = jnp.max(gm, axis=-1, keepdims=True)
    e2 = jnp.min(jnp.where(gm == m2, idx, _E), axis=-1, keepdims=True)
    r = jnp.exp(m2 - m1)
    w_top = 1.0 / (1.0 + r)
    w_sec = r / (1.0 + r)
    c = jnp.where(idx == e1, w_top, 0.0) + jnp.where(idx == e2, w_sec, 0.0)
    h = jnp.dot(x.astype(jnp.bfloat16), w1_ref[...],
                preferred_element_type=jnp.float32)
    h = jnp.maximum(h + b1_ref[...], 0.0)
    cw = jnp.dot(c, emat_ref[...], preferred_element_type=jnp.float32)
    hw = h * cw
    out = jnp.dot(hw, w2_ref[...], preferred_element_type=jnp.float32)
    out_ref[...] = out + jnp.dot(c, b2_ref[...],
                                 preferred_element_type=jnp.float32)


@functools.partial(jax.jit, static_argnames=("interpret",))
def _moe(x, gate_W, gate_b, W1, b1, W2, b2, interpret=False):
    B = x.shape[0]
    w1cat = _relayout_w1(W1)
    b1cat = b1.reshape(1, _E * _DH)
    w2cat = W2.reshape(_E * _DH, _DO)
    emat = jnp.asarray(np.kron(np.eye(_E, dtype=np.float32),
                               np.ones((1, _DH), np.float32)))
    grid = (B // _BLK,)
    out = pl.pallas_call(
        _moe_body,
        grid=grid,
        in_specs=[
            pl.BlockSpec((_BLK, _DM), lambda i: (i, 0)),
            pl.BlockSpec((_DM, _E), lambda i: (0, 0)),
            pl.BlockSpec((1, _E), lambda i: (0, 0)),
            pl.BlockSpec((_DM, _E * _DH), lambda i: (0, 0)),
            pl.BlockSpec((1, _E * _DH), lambda i: (0, 0)),
            pl.BlockSpec((_E * _DH, _DO), lambda i: (0, 0)),
            pl.BlockSpec((_E, _DO), lambda i: (0, 0)),
            pl.BlockSpec((_E, _E * _DH), lambda i: (0, 0)),
        ],
        out_specs=pl.BlockSpec((_BLK, _DO), lambda i: (i, 0)),
        out_shape=jax.ShapeDtypeStruct((B, _DO), jnp.float32),
        interpret=interpret,
    )(x, gate_W, gate_b.reshape(1, _E), w1cat, b1cat, w2cat, b2, emat)
    return out


def kernel(x, gate_W, gate_b, W1, b1, W2, b2):
    return _moe(x, gate_W, gate_b, W1, b1, W2, b2)


# speedup vs baseline: 1.1247x; 1.1014x over previous
import functools

import jax
import jax.numpy as jnp
import numpy as np
from jax.experimental import pallas as pl
from jax.experimental.pallas import tpu as pltpu

_E = 8
_DM = 3072
_DH = 128
_DO = 10
_BLK = 512


def _moe_body(x_ref, gw_ref, gb_ref, w1_ref, b1_ref, w2_ref, b2_ref,
              emat_ref, out_ref, w1s_ref):
    @pl.when(pl.program_id(0) == 0)
    def _relayout():
        for e in range(_E):
            w1s_ref[:, e * _DH:(e + 1) * _DH] = w1_ref[e].astype(jnp.bfloat16)

    x = x_ref[...]
    g = jnp.dot(x, gw_ref[...],
                preferred_element_type=jnp.float32) + gb_ref[...]
    idx = jax.lax.broadcasted_iota(jnp.int32, g.shape, 1)
    m1 = jnp.max(g, axis=-1, keepdims=True)
    e1 = jnp.min(jnp.where(g == m1, idx, _E), axis=-1, keepdims=True)
    gm = jnp.where(idx == e1, -jnp.inf, g)
    m2 = jnp.max(gm, axis=-1, keepdims=True)
    e2 = jnp.min(jnp.where(gm == m2, idx, _E), axis=-1, keepdims=True)
    r = jnp.exp(m2 - m1)
    w_top = 1.0 / (1.0 + r)
    w_sec = r / (1.0 + r)
    c = jnp.where(idx == e1, w_top, 0.0) + jnp.where(idx == e2, w_sec, 0.0)
    h = jnp.dot(x.astype(jnp.bfloat16), w1s_ref[...],
                preferred_element_type=jnp.float32)
    h = jnp.maximum(h + b1_ref[...], 0.0)
    cw = jnp.dot(c, emat_ref[...], preferred_element_type=jnp.float32)
    hw = h * cw
    out = jnp.dot(hw, w2_ref[...], preferred_element_type=jnp.float32)
    out_ref[...] = out + jnp.dot(c, b2_ref[...],
                                 preferred_element_type=jnp.float32)


@functools.partial(jax.jit, static_argnames=("interpret",))
def _moe(x, gate_W, gate_b, W1, b1, W2, b2, interpret=False):
    B = x.shape[0]
    b1cat = b1.reshape(1, _E * _DH)
    w2cat = W2.reshape(_E * _DH, _DO)
    emat = jnp.asarray(np.kron(np.eye(_E, dtype=np.float32),
                               np.ones((1, _DH), np.float32)))
    grid = (B // _BLK,)
    out = pl.pallas_call(
        _moe_body,
        grid=grid,
        in_specs=[
            pl.BlockSpec((_BLK, _DM), lambda i: (i, 0)),
            pl.BlockSpec((_DM, _E), lambda i: (0, 0)),
            pl.BlockSpec((1, _E), lambda i: (0, 0)),
            pl.BlockSpec((_E, _DM, _DH), lambda i: (0, 0, 0)),
            pl.BlockSpec((1, _E * _DH), lambda i: (0, 0)),
            pl.BlockSpec((_E * _DH, _DO), lambda i: (0, 0)),
            pl.BlockSpec((_E, _DO), lambda i: (0, 0)),
            pl.BlockSpec((_E, _E * _DH), lambda i: (0, 0)),
        ],
        out_specs=pl.BlockSpec((_BLK, _DO), lambda i: (i, 0)),
        out_shape=jax.ShapeDtypeStruct((B, _DO), jnp.float32),
        scratch_shapes=[pltpu.VMEM((_DM, _E * _DH), jnp.bfloat16)],
        interpret=interpret,
    )(x, gate_W, gate_b.reshape(1, _E), W1, b1cat, w2cat, b2, emat)
    return out


def kernel(x, gate_W, gate_b, W1, b1, W2, b2):
    return _moe(x, gate_W, gate_b, W1, b1, W2, b2)
